# baseline (speedup 1.0000x reference)
import functools

import jax
import jax.numpy as jnp
from jax.experimental import pallas as pl


def _degree_body(adj_ij, adj_ji, dinv_ref, *, K, T, n):
    i = pl.program_id(0)
    j = pl.program_id(1)
    p = adj_ij[...]
    q = adj_ji[...]
    t = jnp.maximum(p, q.T)
    rows = jax.lax.broadcasted_iota(jnp.int32, (K, K), 0) + i * K
    cols = jax.lax.broadcasted_iota(jnp.int32, (K, K), 1) + j * K
    t = jnp.where((rows < n) & (cols < n), t, 0.0)
    rs = jnp.sum(t, axis=1, keepdims=True)

    @pl.when(j == 0)
    def _():
        dinv_ref[...] = rs

    @pl.when(j > 0)
    def _():
        dinv_ref[...] += rs

    @pl.when(j == T - 1)
    def _():
        d = dinv_ref[...]
        dinv_ref[...] = jnp.where(d > 0, 1.0 / jnp.sqrt(d), 0.0)


def _degree(adj, K, T, n):
    NP = K * T
    body = functools.partial(_degree_body, K=K, T=T, n=n)
    return pl.pallas_call(
        body,
        grid=(T, T),
        in_specs=[
            pl.BlockSpec((K, K), lambda i, j: (i, j)),
            pl.BlockSpec((K, K), lambda i, j: (j, i)),
        ],
        out_specs=pl.BlockSpec((K, 1), lambda i, j: (i, 0)),
        out_shape=jax.ShapeDtypeStruct((NP, 1), jnp.float32),
    )(adj, adj)


def _normalize_body(adj_ij, adj_ji, dinv_i_ref, dinv_j_ref, sn_ref, *, K, n):
    i = pl.program_id(0)
    j = pl.program_id(1)
    p = adj_ij[...]
    q = adj_ji[...]
    t = jnp.maximum(p, q.T)
    rows = jax.lax.broadcasted_iota(jnp.int32, (K, K), 0) + i * K
    cols = jax.lax.broadcasted_iota(jnp.int32, (K, K), 1) + j * K
    t = jnp.where((rows < n) & (cols < n), t, 0.0)
    s = -((t * dinv_i_ref[...]) * dinv_j_ref[...])
    sn_ref[...] = s.astype(jnp.bfloat16)


def _normalize(adj, dinv, dinv_row, K, T, n):
    NP = K * T
    body = functools.partial(_normalize_body, K=K, n=n)
    return pl.pallas_call(
        body,
        grid=(T, T),
        in_specs=[
            pl.BlockSpec((K, K), lambda i, j: (i, j)),
            pl.BlockSpec((K, K), lambda i, j: (j, i)),
            pl.BlockSpec((K, 1), lambda i, j: (i, 0)),
            pl.BlockSpec((1, K), lambda i, j: (0, j)),
        ],
        out_specs=pl.BlockSpec((K, K), lambda i, j: (i, j)),
        out_shape=jax.ShapeDtypeStruct((NP, NP), jnp.bfloat16),
    )(adj, adj, dinv, dinv_row)


def _tri_ij(a, b, T):
    lo = b < T - a
    i = jnp.where(lo, a, T - 1 - a)
    j = jnp.where(lo, a + b, b - 1)
    return i, j


def _matvec_body(a_ref, x_ref, out_ref, *, K, T):
    a = pl.program_id(0)
    b = pl.program_id(1)
    i, j = _tri_ij(a, b, T)

    @pl.when((a == 0) & (b == 0))
    def _():
        out_ref[...] = jnp.zeros_like(out_ref)

    tile = a_ref[...]
    xj = x_ref[pl.ds(j * K, K), :].astype(jnp.bfloat16)
    out_ref[pl.ds(i * K, K), :] += jax.lax.dot_general(
        tile, xj, (((1,), (0,)), ((), ())),
        preferred_element_type=jnp.float32)

    @pl.when(i != j)
    def _():
        xi = x_ref[pl.ds(i * K, K), :].astype(jnp.bfloat16)
        out_ref[pl.ds(j * K, K), :] += jax.lax.dot_general(
            tile, xi, (((0,), (0,)), ((), ())),
            preferred_element_type=jnp.float32)


def _matvec(sn, x, K):
    NP, B = x.shape
    T = NP // K
    body = functools.partial(_matvec_body, K=K, T=T)
    return pl.pallas_call(
        body,
        grid=(T // 2, T + 1),
        in_specs=[
            pl.BlockSpec((K, K), lambda a, b: _tri_ij(a, b, NP // K)),
            pl.BlockSpec((NP, B), lambda a, b: (0, 0)),
        ],
        out_specs=pl.BlockSpec((NP, B), lambda a, b: (0, 0)),
        out_shape=jax.ShapeDtypeStruct((NP, B), jnp.float32),
    )(sn, x)


def _combine_body(x_ref, y1_ref, y2s_ref, co_ref, h_ref):
    def r(v):
        return v.astype(jnp.bfloat16).astype(jnp.float32)

    x = x_ref[...]
    y1 = y1_ref[...]
    y2 = 2.0 * y2s_ref[...] - x
    xb, y1b, y2b = r(x), r(y1), r(y2)
    co = r(co_ref[...])
    g = xb * co[0, 0] + y1b * co[0, 1] + y2b * co[0, 2] + co_ref[0, 3]
    c = xb * co[0, 4] + y1b * co[0, 5] + y2b * co[0, 6] + co_ref[0, 7]
    h_ref[...] = (1.0 - jax.nn.sigmoid(g)) * jnp.tanh(c)


def _combine(x, y1, y2s, co):
    NP, B = x.shape
    return pl.pallas_call(
        _combine_body,
        grid=(1,),
        in_specs=[
            pl.BlockSpec((NP, B), lambda i: (0, 0)),
            pl.BlockSpec((NP, B), lambda i: (0, 0)),
            pl.BlockSpec((NP, B), lambda i: (0, 0)),
            pl.BlockSpec((1, 8), lambda i: (0, 0)),
        ],
        out_specs=pl.BlockSpec((NP, B), lambda i: (0, 0)),
        out_shape=jax.ShapeDtypeStruct((NP, B), jnp.float32),
    )(x, y1, y2s, co)


def _run(inputs, adj, Wg, bg, Wc, bc, K, K2):
    n = adj.shape[0]
    T = -(-n // K)
    NP = K * T
    num_layers = Wg.shape[0]

    dinv = _degree(adj, K, T, n)
    sn = _normalize(adj, dinv, dinv.T, K, T, n)
    x = jnp.pad(inputs.T, ((0, NP - n), (0, 0)))
    hs = []
    for l in range(num_layers):
        y1 = _matvec(sn, x, K)
        y2s = _matvec(sn, y1, K)
        co = jnp.stack([
            Wg[l, 0, 1], Wg[l, 1, 1], Wg[l, 2, 1], bg[l, 1],
            Wc[l, 0, 0], Wc[l, 1, 0], Wc[l, 2, 0], bc[l, 0],
        ]).reshape(1, 8)
        x = _combine(x, y1, y2s, co)
        hs.append(x)

    out = x[:n, :].T
    states = jnp.stack([h[:n, :].T for h in hs], axis=0)
    return out, states


def kernel(inputs, adj, Wg, bg, Wc, bc):
    return _run(inputs, adj, Wg, bg, Wc, bc, K=1024, K2=512)


# speedup vs baseline: 1.0583x; 1.0583x over previous
import functools

import jax
import jax.numpy as jnp
from jax.experimental import pallas as pl
from jax.experimental.pallas import tpu as pltpu


def _degree_body(adj_ij, adj_ji, dinv_ref, *, K, T, n):
    i = pl.program_id(0)
    j = pl.program_id(1)
    p = adj_ij[...]
    q = adj_ji[...]
    t = jnp.maximum(p, q.T)
    rows = jax.lax.broadcasted_iota(jnp.int32, (K, K), 0) + i * K
    cols = jax.lax.broadcasted_iota(jnp.int32, (K, K), 1) + j * K
    t = jnp.where((rows < n) & (cols < n), t, 0.0)
    rs = jnp.sum(t, axis=1, keepdims=True)

    @pl.when(j == 0)
    def _():
        dinv_ref[...] = rs

    @pl.when(j > 0)
    def _():
        dinv_ref[...] += rs

    @pl.when(j == T - 1)
    def _():
        d = dinv_ref[...]
        dinv_ref[...] = jnp.where(d > 0, 1.0 / jnp.sqrt(d), 0.0)


def _degree(adj, K, T, n):
    NP = K * T
    body = functools.partial(_degree_body, K=K, T=T, n=n)
    return pl.pallas_call(
        body,
        grid=(T, T),
        in_specs=[
            pl.BlockSpec((K, K), lambda i, j: (i, j)),
            pl.BlockSpec((K, K), lambda i, j: (j, i)),
        ],
        out_specs=pl.BlockSpec((K, 1), lambda i, j: (i, 0)),
        out_shape=jax.ShapeDtypeStruct((NP, 1), jnp.float32),
    )(adj, adj)


def _normalize_body(adj_ij, adj_ji, dinv_i_ref, dinv_j_ref, sn_ref, *, K, n):
    i = pl.program_id(0)
    j = pl.program_id(1)
    p = adj_ij[...]
    q = adj_ji[...]
    t = jnp.maximum(p, q.T)
    rows = jax.lax.broadcasted_iota(jnp.int32, (K, K), 0) + i * K
    cols = jax.lax.broadcasted_iota(jnp.int32, (K, K), 1) + j * K
    t = jnp.where((rows < n) & (cols < n), t, 0.0)
    s = -((t * dinv_i_ref[...]) * dinv_j_ref[...])
    sn_ref[...] = s.astype(jnp.bfloat16)


def _normalize(adj, dinv, dinv_row, K, T, n):
    NP = K * T
    body = functools.partial(_normalize_body, K=K, n=n)
    return pl.pallas_call(
        body,
        grid=(T, T),
        in_specs=[
            pl.BlockSpec((K, K), lambda i, j: (i, j)),
            pl.BlockSpec((K, K), lambda i, j: (j, i)),
            pl.BlockSpec((K, 1), lambda i, j: (i, 0)),
            pl.BlockSpec((1, K), lambda i, j: (0, j)),
        ],
        out_specs=pl.BlockSpec((K, K), lambda i, j: (i, j)),
        out_shape=jax.ShapeDtypeStruct((NP, NP), jnp.bfloat16),
    )(adj, adj, dinv, dinv_row)


def _tri_ij(a, b, T):
    lo = b < T - a
    i = jnp.where(lo, a, T - 1 - a)
    j = jnp.where(lo, a + b, b - 1)
    return i, j


def _matvec_body(a_ref, x_ref, out_ref, outt_ref, *, K, T):
    a = pl.program_id(0)
    b = pl.program_id(1)
    i, j = _tri_ij(a, b, T)

    @pl.when((a == 0) & (b == 0))
    def _():
        out_ref[...] = jnp.zeros_like(out_ref)
        outt_ref[...] = jnp.zeros_like(outt_ref)

    tile = a_ref[...]
    xj = x_ref[pl.ds(j * K, K), :].astype(jnp.bfloat16)
    out_ref[pl.ds(i * K, K), :] += jax.lax.dot_general(
        tile, xj, (((1,), (0,)), ((), ())),
        preferred_element_type=jnp.float32)

    @pl.when(i != j)
    def _():
        xi = x_ref[pl.ds(i * K, K), :].astype(jnp.bfloat16)
        outt_ref[:, pl.ds(j * K, K)] += jax.lax.dot_general(
            xi, tile, (((0,), (0,)), ((), ())),
            preferred_element_type=jnp.float32)

    @pl.when((a == T // 2 - 1) & (b == T))
    def _():
        out_ref[...] += outt_ref[...].T


def _matvec(sn, x, K):
    NP, B = x.shape
    T = NP // K
    body = functools.partial(_matvec_body, K=K, T=T)
    return pl.pallas_call(
        body,
        grid=(T // 2, T + 1),
        in_specs=[
            pl.BlockSpec((K, K), lambda a, b: _tri_ij(a, b, NP // K)),
            pl.BlockSpec((NP, B), lambda a, b: (0, 0)),
        ],
        out_specs=pl.BlockSpec((NP, B), lambda a, b: (0, 0)),
        out_shape=jax.ShapeDtypeStruct((NP, B), jnp.float32),
        scratch_shapes=[pltpu.VMEM((B, NP), jnp.float32)],
    )(sn, x)


def _combine_body(x_ref, y1_ref, y2s_ref, co_ref, h_ref):
    def r(v):
        return v.astype(jnp.bfloat16).astype(jnp.float32)

    x = x_ref[...]
    y1 = y1_ref[...]
    y2 = 2.0 * y2s_ref[...] - x
    xb, y1b, y2b = r(x), r(y1), r(y2)
    co = r(co_ref[...])
    g = xb * co[0, 0] + y1b * co[0, 1] + y2b * co[0, 2] + co_ref[0, 3]
    c = xb * co[0, 4] + y1b * co[0, 5] + y2b * co[0, 6] + co_ref[0, 7]
    h_ref[...] = (1.0 - jax.nn.sigmoid(g)) * jnp.tanh(c)


def _combine(x, y1, y2s, co):
    NP, B = x.shape
    return pl.pallas_call(
        _combine_body,
        grid=(1,),
        in_specs=[
            pl.BlockSpec((NP, B), lambda i: (0, 0)),
            pl.BlockSpec((NP, B), lambda i: (0, 0)),
            pl.BlockSpec((NP, B), lambda i: (0, 0)),
            pl.BlockSpec((1, 8), lambda i: (0, 0)),
        ],
        out_specs=pl.BlockSpec((NP, B), lambda i: (0, 0)),
        out_shape=jax.ShapeDtypeStruct((NP, B), jnp.float32),
    )(x, y1, y2s, co)


def _run(inputs, adj, Wg, bg, Wc, bc, K, K2):
    n = adj.shape[0]
    T = -(-n // K)
    NP = K * T
    num_layers = Wg.shape[0]

    dinv = _degree(adj, K, T, n)
    sn = _normalize(adj, dinv, dinv.T, K, T, n)
    x = jnp.pad(inputs.T, ((0, NP - n), (0, 0)))
    hs = []
    for l in range(num_layers):
        y1 = _matvec(sn, x, K)
        y2s = _matvec(sn, y1, K)
        co = jnp.stack([
            Wg[l, 0, 1], Wg[l, 1, 1], Wg[l, 2, 1], bg[l, 1],
            Wc[l, 0, 0], Wc[l, 1, 0], Wc[l, 2, 0], bc[l, 0],
        ]).reshape(1, 8)
        x = _combine(x, y1, y2s, co)
        hs.append(x)

    out = x[:n, :].T
    states = jnp.stack([h[:n, :].T for h in hs], axis=0)
    return out, states


def kernel(inputs, adj, Wg, bg, Wc, bc):
    return _run(inputs, adj, Wg, bg, Wc, bc, K=1024, K2=512)


# speedup vs baseline: 1.0838x; 1.0241x over previous
import functools

import jax
import jax.numpy as jnp
from jax.experimental import pallas as pl


def _tri_ij(a, b, T):
    lo = b < T - a
    i = jnp.where(lo, a, T - 1 - a)
    j = jnp.where(lo, a + b, b - 1)
    return i, j


def _masked_sym_tile(p, q, row_base, col_base, K, n):
    t = jnp.maximum(p, q.T)
    rows = jax.lax.broadcasted_iota(jnp.int32, (K, K), 0) + row_base
    cols = jax.lax.broadcasted_iota(jnp.int32, (K, K), 1) + col_base
    return jnp.where((rows < n) & (cols < n), t, 0.0)


def _degree_body(adj_ij, adj_ji, dinv_ref, *, K, T, n):
    a = pl.program_id(0)
    b = pl.program_id(1)
    i, j = _tri_ij(a, b, T)

    @pl.when((a == 0) & (b == 0))
    def _():
        dinv_ref[...] = jnp.zeros_like(dinv_ref)

    t = _masked_sym_tile(adj_ij[...], adj_ji[...], i * K, j * K, K, n)
    dinv_ref[pl.ds(i * K, K), :] += jnp.sum(t, axis=1, keepdims=True)

    @pl.when(i != j)
    def _():
        ones = jnp.ones((K, 1), jnp.float32)
        cs = jax.lax.dot_general(
            t, ones, (((0,), (0,)), ((), ())),
            preferred_element_type=jnp.float32,
            precision=jax.lax.Precision.HIGHEST)
        dinv_ref[pl.ds(j * K, K), :] += cs

    @pl.when((a == T // 2 - 1) & (b == T))
    def _():
        d = dinv_ref[...]
        dinv_ref[...] = jnp.where(d > 0, 1.0 / jnp.sqrt(d), 0.0)


def _degree(adj, K, T, n):
    NP = K * T
    body = functools.partial(_degree_body, K=K, T=T, n=n)
    return pl.pallas_call(
        body,
        grid=(T // 2, T + 1),
        in_specs=[
            pl.BlockSpec((K, K), lambda a, b: _tri_ij(a, b, NP // K)),
            pl.BlockSpec((K, K), lambda a, b: _tri_ij(a, b, NP // K)[::-1]),
        ],
        out_specs=pl.BlockSpec((NP, 1), lambda a, b: (0, 0)),
        out_shape=jax.ShapeDtypeStruct((NP, 1), jnp.float32),
    )(adj, adj)


def _normalize_body(adj_ij, adj_ji, dinv_c_ref, dinv_r_ref, sn_ref, *, K, T, n):
    a = pl.program_id(0)
    b = pl.program_id(1)
    s = pl.program_id(2)
    i, j = _tri_ij(a, b, T)

    @pl.when(s == 0)
    def _():
        t = _masked_sym_tile(adj_ij[...], adj_ji[...], i * K, j * K, K, n)
        sn_ref[...] = (-((t * dinv_c_ref[...]) * dinv_r_ref[...])).astype(
            jnp.bfloat16)

    @pl.when(s == 1)
    def _():
        t = _masked_sym_tile(adj_ji[...], adj_ij[...], j * K, i * K, K, n)
        sn_ref[...] = (-((t * dinv_c_ref[...]) * dinv_r_ref[...])).astype(
            jnp.bfloat16)


def _normalize(adj, dinv, dinv_row, K, T, n):
    NP = K * T

    def _swap(a, b, s):
        i, j = _tri_ij(a, b, T)
        io = jnp.where(s == 0, i, j)
        jo = jnp.where(s == 0, j, i)
        return io, jo

    body = functools.partial(_normalize_body, K=K, T=T, n=n)
    return pl.pallas_call(
        body,
        grid=(T // 2, T + 1, 2),
        in_specs=[
            pl.BlockSpec((K, K), lambda a, b, s: _tri_ij(a, b, T)),
            pl.BlockSpec((K, K), lambda a, b, s: _tri_ij(a, b, T)[::-1]),
            pl.BlockSpec((K, 1), lambda a, b, s: (_swap(a, b, s)[0], 0)),
            pl.BlockSpec((1, K), lambda a, b, s: (0, _swap(a, b, s)[1])),
        ],
        out_specs=pl.BlockSpec((K, K), lambda a, b, s: _swap(a, b, s)),
        out_shape=jax.ShapeDtypeStruct((NP, NP), jnp.bfloat16),
    )(adj, adj, dinv, dinv_row)


def _matvec_body(a_ref, x_ref, out_ref):
    xb = x_ref[...].astype(jnp.bfloat16)
    out_ref[...] = jax.lax.dot_general(
        a_ref[...], xb, (((1,), (0,)), ((), ())),
        preferred_element_type=jnp.float32)


def _matvec(sn, x, K2):
    NP, B = x.shape
    T2 = NP // K2
    return pl.pallas_call(
        _matvec_body,
        grid=(T2,),
        in_specs=[
            pl.BlockSpec((K2, NP), lambda i: (i, 0)),
            pl.BlockSpec((NP, B), lambda i: (0, 0)),
        ],
        out_specs=pl.BlockSpec((K2, B), lambda i: (i, 0)),
        out_shape=jax.ShapeDtypeStruct((NP, B), jnp.float32),
    )(sn, x)


def _matvec_combine_body(a_ref, y1_ref, x_ref, co_ref, h_ref, *, K2):
    i = pl.program_id(0)
    y1b = y1_ref[...].astype(jnp.bfloat16)
    y2s = jax.lax.dot_general(
        a_ref[...], y1b, (((1,), (0,)), ((), ())),
        preferred_element_type=jnp.float32)
    x = x_ref[...]
    y2 = 2.0 * y2s - x
    xb = x.astype(jnp.bfloat16).astype(jnp.float32)
    y1bf = y1_ref[pl.ds(i * K2, K2), :].astype(jnp.bfloat16).astype(
        jnp.float32)
    y2b = y2.astype(jnp.bfloat16).astype(jnp.float32)
    co = co_ref[...].astype(jnp.bfloat16).astype(jnp.float32)
    g = xb * co[0, 0] + y1bf * co[0, 1] + y2b * co[0, 2] + co_ref[0, 3]
    c = xb * co[0, 4] + y1bf * co[0, 5] + y2b * co[0, 6] + co_ref[0, 7]
    h_ref[...] = (1.0 - jax.nn.sigmoid(g)) * jnp.tanh(c)


def _matvec_combine(sn, y1, x, co, K2):
    NP, B = x.shape
    T2 = NP // K2
    body = functools.partial(_matvec_combine_body, K2=K2)
    return pl.pallas_call(
        body,
        grid=(T2,),
        in_specs=[
            pl.BlockSpec((K2, NP), lambda i: (i, 0)),
            pl.BlockSpec((NP, B), lambda i: (0, 0)),
            pl.BlockSpec((K2, B), lambda i: (i, 0)),
            pl.BlockSpec((1, 8), lambda i: (0, 0)),
        ],
        out_specs=pl.BlockSpec((K2, B), lambda i: (i, 0)),
        out_shape=jax.ShapeDtypeStruct((NP, B), jnp.float32),
    )(sn, y1, x, co)


def _run(inputs, adj, Wg, bg, Wc, bc, K, K2):
    n = adj.shape[0]
    T = -(-n // K)
    NP = K * T
    num_layers = Wg.shape[0]

    dinv = _degree(adj, K, T, n)
    sn = _normalize(adj, dinv, dinv.T, K, T, n)
    x = jnp.pad(inputs.T, ((0, NP - n), (0, 0)))
    hs = []
    for l in range(num_layers):
        y1 = _matvec(sn, x, K2)
        co = jnp.stack([
            Wg[l, 0, 1], Wg[l, 1, 1], Wg[l, 2, 1], bg[l, 1],
            Wc[l, 0, 0], Wc[l, 1, 0], Wc[l, 2, 0], bc[l, 0],
        ]).reshape(1, 8)
        x = _matvec_combine(sn, y1, x, co, K2)
        hs.append(x)

    out = x[:n, :].T
    states = jnp.stack([h[:n, :].T for h in hs], axis=0)
    return out, states


def kernel(inputs, adj, Wg, bg, Wc, bc):
    return _run(inputs, adj, Wg, bg, Wc, bc, K=1024, K2=1024)


# speedup vs baseline: 1.2536x; 1.1567x over previous
import functools

import jax
import jax.numpy as jnp
from jax.experimental import pallas as pl
from jax.experimental.pallas import tpu as pltpu


def _tri_ij(a, b, T):
    lo = b < T - a
    i = jnp.where(lo, a, T - 1 - a)
    j = jnp.where(lo, a + b, b - 1)
    return i, j


def _masked_sym_tile(p, q, row_base, col_base, K, n):
    t = jnp.maximum(p, q.T)
    rows = jax.lax.broadcasted_iota(jnp.int32, (K, K), 0) + row_base
    cols = jax.lax.broadcasted_iota(jnp.int32, (K, K), 1) + col_base
    return jnp.where((rows < n) & (cols < n), t, 0.0)


def _degree_body(adj_ij, adj_ji, dinv_ref, cs_ref, *, K, T, n):
    a = pl.program_id(0)
    b = pl.program_id(1)
    i, j = _tri_ij(a, b, T)

    @pl.when((a == 0) & (b == 0))
    def _():
        dinv_ref[...] = jnp.zeros_like(dinv_ref)
        cs_ref[...] = jnp.zeros_like(cs_ref)

    t = _masked_sym_tile(adj_ij[...], adj_ji[...], i * K, j * K, K, n)
    dinv_ref[pl.ds(i * K, K), :] += jnp.sum(t, axis=1, keepdims=True)

    @pl.when(i != j)
    def _():
        cs_ref[:, pl.ds(j * K, K)] += jnp.sum(t, axis=0, keepdims=True)

    @pl.when((a == T // 2 - 1) & (b == T))
    def _():
        d = dinv_ref[...] + cs_ref[...].T
        dinv_ref[...] = jnp.where(d > 0, 1.0 / jnp.sqrt(d), 0.0)


def _degree(adj, K, T, n):
    NP = K * T
    body = functools.partial(_degree_body, K=K, T=T, n=n)
    return pl.pallas_call(
        body,
        grid=(T // 2, T + 1),
        in_specs=[
            pl.BlockSpec((K, K), lambda a, b: _tri_ij(a, b, NP // K)),
            pl.BlockSpec((K, K), lambda a, b: _tri_ij(a, b, NP // K)[::-1]),
        ],
        out_specs=pl.BlockSpec((NP, 1), lambda a, b: (0, 0)),
        out_shape=jax.ShapeDtypeStruct((NP, 1), jnp.float32),
        scratch_shapes=[pltpu.VMEM((1, NP), jnp.float32)],
    )(adj, adj)


def _normalize_body(adj_ij, adj_ji, dinv_c_ref, dinv_r_ref, sn_ref, *, K, n):
    i = pl.program_id(0)
    j = pl.program_id(1)
    t = _masked_sym_tile(adj_ij[...], adj_ji[...], i * K, j * K, K, n)
    sn_ref[...] = (-((t * dinv_c_ref[...]) * dinv_r_ref[...])).astype(
        jnp.bfloat16)


def _normalize(adj, dinv, dinv_row, K, T, n):
    NP = K * T
    body = functools.partial(_normalize_body, K=K, n=n)
    return pl.pallas_call(
        body,
        grid=(T, T),
        in_specs=[
            pl.BlockSpec((K, K), lambda i, j: (i, j)),
            pl.BlockSpec((K, K), lambda i, j: (j, i)),
            pl.BlockSpec((K, 1), lambda i, j: (i, 0)),
            pl.BlockSpec((1, K), lambda i, j: (0, j)),
        ],
        out_specs=pl.BlockSpec((K, K), lambda i, j: (i, j)),
        out_shape=jax.ShapeDtypeStruct((NP, NP), jnp.bfloat16),
    )(adj, adj, dinv, dinv_row)


def _matvec_body(a_ref, x_ref, out_ref):
    xb = x_ref[...].astype(jnp.bfloat16)
    out_ref[...] = jax.lax.dot_general(
        a_ref[...], xb, (((1,), (0,)), ((), ())),
        preferred_element_type=jnp.float32)


def _matvec(sn, x, K2):
    NP, B = x.shape
    T2 = NP // K2
    return pl.pallas_call(
        _matvec_body,
        grid=(T2,),
        in_specs=[
            pl.BlockSpec((K2, NP), lambda i: (i, 0)),
            pl.BlockSpec((NP, B), lambda i: (0, 0)),
        ],
        out_specs=pl.BlockSpec((K2, B), lambda i: (i, 0)),
        out_shape=jax.ShapeDtypeStruct((NP, B), jnp.float32),
    )(sn, x)


def _matvec_combine_body(a_ref, y1_ref, x_ref, co_ref, h_ref, *, K2):
    i = pl.program_id(0)
    y1b = y1_ref[...].astype(jnp.bfloat16)
    y2s = jax.lax.dot_general(
        a_ref[...], y1b, (((1,), (0,)), ((), ())),
        preferred_element_type=jnp.float32)
    x = x_ref[...]
    y2 = 2.0 * y2s - x
    xb = x.astype(jnp.bfloat16).astype(jnp.float32)
    y1bf = y1_ref[pl.ds(i * K2, K2), :].astype(jnp.bfloat16).astype(
        jnp.float32)
    y2b = y2.astype(jnp.bfloat16).astype(jnp.float32)
    co = co_ref[...].astype(jnp.bfloat16).astype(jnp.float32)
    g = xb * co[0, 0] + y1bf * co[0, 1] + y2b * co[0, 2] + co_ref[0, 3]
    c = xb * co[0, 4] + y1bf * co[0, 5] + y2b * co[0, 6] + co_ref[0, 7]
    h_ref[...] = (1.0 - jax.nn.sigmoid(g)) * jnp.tanh(c)


def _matvec_combine(sn, y1, x, co, K2):
    NP, B = x.shape
    T2 = NP // K2
    body = functools.partial(_matvec_combine_body, K2=K2)
    return pl.pallas_call(
        body,
        grid=(T2,),
        in_specs=[
            pl.BlockSpec((K2, NP), lambda i: (i, 0)),
            pl.BlockSpec((NP, B), lambda i: (0, 0)),
            pl.BlockSpec((K2, B), lambda i: (i, 0)),
            pl.BlockSpec((1, 8), lambda i: (0, 0)),
        ],
        out_specs=pl.BlockSpec((K2, B), lambda i: (i, 0)),
        out_shape=jax.ShapeDtypeStruct((NP, B), jnp.float32),
    )(sn, y1, x, co)


def _run(inputs, adj, Wg, bg, Wc, bc, K, K2):
    n = adj.shape[0]
    T = -(-n // K)
    NP = K * T
    num_layers = Wg.shape[0]

    dinv = _degree(adj, K, T, n)
    sn = _normalize(adj, dinv, dinv.T, K, T, n)
    x = jnp.pad(inputs.T, ((0, NP - n), (0, 0)))
    hs = []
    for l in range(num_layers):
        y1 = _matvec(sn, x, K2)
        co = jnp.stack([
            Wg[l, 0, 1], Wg[l, 1, 1], Wg[l, 2, 1], bg[l, 1],
            Wc[l, 0, 0], Wc[l, 1, 0], Wc[l, 2, 0], bc[l, 0],
        ]).reshape(1, 8)
        x = _matvec_combine(sn, y1, x, co, K2)
        hs.append(x)

    out = x[:n, :].T
    states = jnp.stack([h[:n, :].T for h in hs], axis=0)
    return out, states


def kernel(inputs, adj, Wg, bg, Wc, bc):
    return _run(inputs, adj, Wg, bg, Wc, bc, K=1024, K2=512)


# speedup vs baseline: 1.5360x; 1.2252x over previous
import functools

import jax
import jax.numpy as jnp
from jax.experimental import pallas as pl
from jax.experimental.pallas import tpu as pltpu


def _tri_ij(a, b, T):
    lo = b < T - a
    i = jnp.where(lo, a, T - 1 - a)
    j = jnp.where(lo, a + b, b - 1)
    return i, j


def _masked_sym_tile(p, q, row_base, col_base, K, n):
    t = jnp.maximum(p, q.T)
    rows = jax.lax.broadcasted_iota(jnp.int32, (K, K), 0) + row_base
    cols = jax.lax.broadcasted_iota(jnp.int32, (K, K), 1) + col_base
    return jnp.where((rows < n) & (cols < n), t, 0.0)


def _degree_body(adj_ij, adj_ji, dinv_ref, cs_ref, *, K, T, n):
    a = pl.program_id(0)
    b = pl.program_id(1)
    i, j = _tri_ij(a, b, T)

    @pl.when((a == 0) & (b == 0))
    def _():
        dinv_ref[...] = jnp.zeros_like(dinv_ref)
        cs_ref[...] = jnp.zeros_like(cs_ref)

    t = _masked_sym_tile(adj_ij[...], adj_ji[...], i * K, j * K, K, n)
    dinv_ref[pl.ds(i * K, K), :] += jnp.sum(t, axis=1, keepdims=True)

    @pl.when(i != j)
    def _():
        cs_ref[:, pl.ds(j * K, K)] += jnp.sum(t, axis=0, keepdims=True)

    @pl.when((a == T // 2 - 1) & (b == T))
    def _():
        d = dinv_ref[...] + cs_ref[...].T
        dinv_ref[...] = jnp.where(d > 0, 1.0 / jnp.sqrt(d), 0.0)


def _degree(adj, K, T, n):
    NP = K * T
    body = functools.partial(_degree_body, K=K, T=T, n=n)
    return pl.pallas_call(
        body,
        grid=(T // 2, T + 1),
        in_specs=[
            pl.BlockSpec((K, K), lambda a, b: _tri_ij(a, b, NP // K)),
            pl.BlockSpec((K, K), lambda a, b: _tri_ij(a, b, NP // K)[::-1]),
        ],
        out_specs=pl.BlockSpec((NP, 1), lambda a, b: (0, 0)),
        out_shape=jax.ShapeDtypeStruct((NP, 1), jnp.float32),
        scratch_shapes=[pltpu.VMEM((1, NP), jnp.float32)],
    )(adj, adj)


def _normalize_body(adj_ij, adj_ji, dinv_c_ref, dinv_r_ref, sn_ref, *, K, T, n):
    a = pl.program_id(0)
    b = pl.program_id(1)
    i, j = _tri_ij(a, b, T)
    t = _masked_sym_tile(adj_ij[...], adj_ji[...], i * K, j * K, K, n)
    sn_ref[...] = (-((t * dinv_c_ref[...]) * dinv_r_ref[...])).astype(
        jnp.bfloat16)


def _normalize(adj, dinv, dinv_row, K, T, n):
    NP = K * T
    body = functools.partial(_normalize_body, K=K, T=T, n=n)
    return pl.pallas_call(
        body,
        grid=(T // 2, T + 1),
        in_specs=[
            pl.BlockSpec((K, K), lambda a, b: _tri_ij(a, b, T)),
            pl.BlockSpec((K, K), lambda a, b: _tri_ij(a, b, T)[::-1]),
            pl.BlockSpec((K, 1), lambda a, b: (_tri_ij(a, b, T)[0], 0)),
            pl.BlockSpec((1, K), lambda a, b: (0, _tri_ij(a, b, T)[1])),
        ],
        out_specs=pl.BlockSpec((K, K), lambda a, b: _tri_ij(a, b, T)),
        out_shape=jax.ShapeDtypeStruct((NP, NP), jnp.bfloat16),
    )(adj, adj, dinv, dinv_row)


def _tri_mv_step(a_ref, x_ref, out_ref, outt_ref, K, T):
    a = pl.program_id(0)
    b = pl.program_id(1)
    i, j = _tri_ij(a, b, T)

    @pl.when((a == 0) & (b == 0))
    def _():
        out_ref[...] = jnp.zeros_like(out_ref)
        outt_ref[...] = jnp.zeros_like(outt_ref)

    tile = a_ref[...]
    xj = x_ref[pl.ds(j * K, K), :].astype(jnp.bfloat16)
    out_ref[pl.ds(i * K, K), :] += jax.lax.dot_general(
        tile, xj, (((1,), (0,)), ((), ())),
        preferred_element_type=jnp.float32)

    @pl.when(i != j)
    def _():
        xi = x_ref[pl.ds(i * K, K), :].astype(jnp.bfloat16)
        outt_ref[:, pl.ds(j * K, K)] += jax.lax.dot_general(
            xi, tile, (((0,), (0,)), ((), ())),
            preferred_element_type=jnp.float32)

    return (a == T // 2 - 1) & (b == T)


def _matvec_body(a_ref, x_ref, out_ref, outt_ref, *, K, T):
    last = _tri_mv_step(a_ref, x_ref, out_ref, outt_ref, K, T)

    @pl.when(last)
    def _():
        out_ref[...] += outt_ref[...].T


def _matvec(sn, x, K):
    NP, B = x.shape
    T = NP // K
    body = functools.partial(_matvec_body, K=K, T=T)
    return pl.pallas_call(
        body,
        grid=(T // 2, T + 1),
        in_specs=[
            pl.BlockSpec((K, K), lambda a, b: _tri_ij(a, b, NP // K)),
            pl.BlockSpec((NP, B), lambda a, b: (0, 0)),
        ],
        out_specs=pl.BlockSpec((NP, B), lambda a, b: (0, 0)),
        out_shape=jax.ShapeDtypeStruct((NP, B), jnp.float32),
        scratch_shapes=[pltpu.VMEM((B, NP), jnp.float32)],
    )(sn, x)


def _matvec_combine_body(a_ref, y1_ref, x_ref, co_ref, h_ref, outt_ref,
                         *, K, T):
    last = _tri_mv_step(a_ref, y1_ref, h_ref, outt_ref, K, T)

    @pl.when(last)
    def _():
        y2s = h_ref[...] + outt_ref[...].T
        x = x_ref[...]
        y2 = 2.0 * y2s - x
        xb = x.astype(jnp.bfloat16).astype(jnp.float32)
        y1b = y1_ref[...].astype(jnp.bfloat16).astype(jnp.float32)
        y2b = y2.astype(jnp.bfloat16).astype(jnp.float32)
        co = co_ref[...].astype(jnp.bfloat16).astype(jnp.float32)
        g = xb * co[0, 0] + y1b * co[0, 1] + y2b * co[0, 2] + co_ref[0, 3]
        c = xb * co[0, 4] + y1b * co[0, 5] + y2b * co[0, 6] + co_ref[0, 7]
        h_ref[...] = (1.0 - jax.nn.sigmoid(g)) * jnp.tanh(c)


def _matvec_combine(sn, y1, x, co, K):
    NP, B = x.shape
    T = NP // K
    body = functools.partial(_matvec_combine_body, K=K, T=T)
    return pl.pallas_call(
        body,
        grid=(T // 2, T + 1),
        in_specs=[
            pl.BlockSpec((K, K), lambda a, b: _tri_ij(a, b, NP // K)),
            pl.BlockSpec((NP, B), lambda a, b: (0, 0)),
            pl.BlockSpec((NP, B), lambda a, b: (0, 0)),
            pl.BlockSpec((1, 8), lambda a, b: (0, 0)),
        ],
        out_specs=pl.BlockSpec((NP, B), lambda a, b: (0, 0)),
        out_shape=jax.ShapeDtypeStruct((NP, B), jnp.float32),
        scratch_shapes=[pltpu.VMEM((B, NP), jnp.float32)],
    )(sn, y1, x, co)


def _run(inputs, adj, Wg, bg, Wc, bc, K, K2):
    n = adj.shape[0]
    T = -(-n // K)
    NP = K * T
    num_layers = Wg.shape[0]

    dinv = _degree(adj, K, T, n)
    sn = _normalize(adj, dinv, dinv.T, K, T, n)
    x = jnp.pad(inputs.T, ((0, NP - n), (0, 0)))
    hs = []
    for l in range(num_layers):
        y1 = _matvec(sn, x, K)
        co = jnp.stack([
            Wg[l, 0, 1], Wg[l, 1, 1], Wg[l, 2, 1], bg[l, 1],
            Wc[l, 0, 0], Wc[l, 1, 0], Wc[l, 2, 0], bc[l, 0],
        ]).reshape(1, 8)
        x = _matvec_combine(sn, y1, x, co, K)
        hs.append(x)

    out = x[:n, :].T
    states = jnp.stack([h[:n, :].T for h in hs], axis=0)
    return out, states


def kernel(inputs, adj, Wg, bg, Wc, bc):
    return _run(inputs, adj, Wg, bg, Wc, bc, K=1280, K2=512)
